# baseline (speedup 1.0000x reference)
import functools

import jax
import jax.numpy as jnp
from jax import lax
from jax.experimental import pallas as pl
from jax.experimental.pallas import tpu as pltpu
from jax.experimental.pallas import tpu_sc as plsc

N = 10000
D = 128
DH = D // 2
E = 320000

NC = 2
NS = 16
CHUNK = 200
NCHUNK = 100
NBUF = 3
ROWS_PER_TILE = N // NS
ZROWS = 25
ZBLKS = ROWS_PER_TILE // ZROWS


def _matmul(x, w):
    BM = 2000

    def body(x_ref, w_ref, o0_ref, o1_ref):
        s = jnp.dot(x_ref[...], w_ref[...], preferred_element_type=jnp.float32)
        o0_ref[...] = s[:, :DH]
        o1_ref[...] = s[:, DH:]

    return pl.pallas_call(
        body,
        grid=(N // BM,),
        in_specs=[pl.BlockSpec((BM, D), lambda i: (i, 0)),
                  pl.BlockSpec((D, D), lambda i: (0, 0))],
        out_specs=[pl.BlockSpec((BM, DH), lambda i: (i, 0)),
                   pl.BlockSpec((BM, DH), lambda i: (i, 0))],
        out_shape=[jax.ShapeDtypeStruct((N, DH), jnp.float32),
                   jax.ShapeDtypeStruct((N, DH), jnp.float32)],
    )(x, w)


def _scatter_body(ei_hbm, sup0_hbm, sup1_hbm, bias_hbm, out_hbm,
                  cb, rb, bias_v, rows, zbuf, accum, gsems, ssems):
    c = 1 - lax.axis_index("c")
    s = lax.axis_index("s")

    pltpu.sync_copy(ei_hbm.at[0, pl.ds(s * NCHUNK, NCHUNK)], rb)
    pltpu.sync_copy(ei_hbm.at[1, pl.ds(s * NCHUNK, NCHUNK)], cb)
    pltpu.sync_copy(bias_hbm, bias_v)

    def zrow(r, carry):
        for j in range(DH // 16):
            zbuf[r, pl.ds(j * 16, 16)] = bias_v[pl.ds(c * DH + j * 16, 16)]
        return carry

    lax.fori_loop(0, ZROWS, zrow, 0)
    for b in range(ZBLKS):
        pltpu.sync_copy(
            zbuf, accum.at[pl.ds(s * ROWS_PER_TILE + b * ZROWS, ZROWS)])

    plsc.subcore_barrier()

    def run(sup_hbm):
        def fire_gather(g, b):
            pltpu.async_copy(sup_hbm.at[cb.at[g]], rows.at[b], gsems.at[b])

        def wait_gather(g, b):
            pltpu.make_async_copy(
                sup_hbm.at[cb.at[g]], rows.at[b], gsems.at[b]).wait()

        def fire_scatter(g, b):
            pltpu.async_copy(rows.at[b], accum.at[rb.at[g]], ssems.at[b],
                             add=True)

        def wait_scatter(g, b):
            pltpu.make_async_copy(
                rows.at[b], accum.at[rb.at[g]], ssems.at[b]).wait()

        for b in range(NBUF):
            fire_gather(b, b)

        NFULL = NCHUNK // NBUF - 1

        def body(i, carry):
            g0 = i * NBUF
            for b in range(NBUF):
                wait_gather(g0 + b, b)
                fire_scatter(g0 + b, b)
            for b in range(NBUF):
                wait_scatter(g0 + b, b)
                nxt = g0 + NBUF + b

                @pl.when(nxt < NCHUNK)
                def _():
                    fire_gather(nxt, b)
            return carry

        lax.fori_loop(0, NFULL + 1, body, 0)
        for b in range(NCHUNK - (NFULL + 1) * NBUF):
            g = (NFULL + 1) * NBUF + b
            wait_gather(g, b)
            fire_scatter(g, b)
            wait_scatter(g, b)

    @pl.when(c == 0)
    def _():
        run(sup0_hbm)

    @pl.when(c == 1)
    def _():
        run(sup1_hbm)

    plsc.subcore_barrier()

    r0 = s * ROWS_PER_TILE
    pltpu.sync_copy(accum.at[pl.ds(r0, ROWS_PER_TILE)],
                    out_hbm.at[pl.ds(r0, ROWS_PER_TILE), pl.ds(c * DH, DH)])


def _scatter(ei3, sup0, sup1, bias):
    mesh = plsc.VectorSubcoreMesh(core_axis_name="c", subcore_axis_name="s")
    k = functools.partial(
        pl.kernel,
        out_type=jax.ShapeDtypeStruct((N, D), jnp.float32),
        mesh=mesh,
        scratch_types=[
            pltpu.VMEM((NCHUNK, CHUNK), jnp.int32),
            pltpu.VMEM((NCHUNK, CHUNK), jnp.int32),
            pltpu.VMEM((D,), jnp.float32),
            pltpu.VMEM((NBUF, CHUNK, DH), jnp.float32),
            pltpu.VMEM((ZROWS, DH), jnp.float32),
            pltpu.VMEM_SHARED((N, DH), jnp.float32),
            pltpu.SemaphoreType.DMA((NBUF,)),
            pltpu.SemaphoreType.DMA((NBUF,)),
        ],
        compiler_params=pltpu.CompilerParams(use_tc_tiling_on_sc=False),
    )(_scatter_body)
    return k(ei3, sup0, sup1, bias)


def kernel(edge_index, input_feature, weight, bias):
    sup0, sup1 = _matmul(input_feature, weight)
    return _scatter(edge_index.reshape(2, NS * NCHUNK, CHUNK),
                    sup0, sup1, bias)


# speedup vs baseline: 1.0497x; 1.0497x over previous
import functools

import jax
import jax.numpy as jnp
from jax import lax
from jax.experimental import pallas as pl
from jax.experimental.pallas import tpu as pltpu
from jax.experimental.pallas import tpu_sc as plsc

N = 10000
D = 128
DH = D // 2
E = 320000

NC = 2
NS = 16
CHUNK = 160
NCHUNK = 125
NBUF = 4
ROWS_PER_TILE = N // NS
ZROWS = 25
ZBLKS = ROWS_PER_TILE // ZROWS


def _matmul(x, w):
    BM = 2000

    def body(x_ref, w_ref, o0_ref, o1_ref):
        s = jnp.dot(x_ref[...], w_ref[...], preferred_element_type=jnp.float32)
        o0_ref[...] = s[:, :DH]
        o1_ref[...] = s[:, DH:]

    return pl.pallas_call(
        body,
        grid=(N // BM,),
        in_specs=[pl.BlockSpec((BM, D), lambda i: (i, 0)),
                  pl.BlockSpec((D, D), lambda i: (0, 0))],
        out_specs=[pl.BlockSpec((BM, DH), lambda i: (i, 0)),
                   pl.BlockSpec((BM, DH), lambda i: (i, 0))],
        out_shape=[jax.ShapeDtypeStruct((N, DH), jnp.float32),
                   jax.ShapeDtypeStruct((N, DH), jnp.float32)],
    )(x, w)


def _scatter_body(ei_hbm, sup0_hbm, sup1_hbm, bias_hbm, out_hbm,
                  cb, rb, bias_v, rows, zbuf, accum, gsems, ssems):
    c = 1 - lax.axis_index("c")
    s = lax.axis_index("s")

    pltpu.sync_copy(ei_hbm.at[0, pl.ds(s * NCHUNK, NCHUNK)], rb)
    pltpu.sync_copy(ei_hbm.at[1, pl.ds(s * NCHUNK, NCHUNK)], cb)
    pltpu.sync_copy(bias_hbm, bias_v)

    def zrow(r, carry):
        for j in range(DH // 16):
            zbuf[r, pl.ds(j * 16, 16)] = bias_v[pl.ds(c * DH + j * 16, 16)]
        return carry

    lax.fori_loop(0, ZROWS, zrow, 0)
    for b in range(ZBLKS):
        pltpu.sync_copy(
            zbuf, accum.at[pl.ds(s * ROWS_PER_TILE + b * ZROWS, ZROWS)])

    plsc.subcore_barrier()

    def run(sup_hbm):
        def fire_gather(g, b):
            pltpu.async_copy(sup_hbm.at[cb.at[g]], rows.at[b], gsems.at[b])

        def wait_gather(g, b):
            pltpu.make_async_copy(
                sup_hbm.at[cb.at[g]], rows.at[b], gsems.at[b]).wait()

        def fire_scatter(g, b):
            pltpu.async_copy(rows.at[b], accum.at[rb.at[g]], ssems.at[b],
                             add=True)

        def wait_scatter(g, b):
            pltpu.make_async_copy(
                rows.at[b], accum.at[rb.at[g]], ssems.at[b]).wait()

        for b in range(NBUF):
            fire_gather(b, b)

        NFULL = NCHUNK // NBUF - 1

        def body(i, carry):
            g0 = i * NBUF
            for b in range(NBUF):
                wait_gather(g0 + b, b)
                fire_scatter(g0 + b, b)
            for b in range(NBUF):
                wait_scatter(g0 + b, b)
                nxt = g0 + NBUF + b

                @pl.when(nxt < NCHUNK)
                def _():
                    fire_gather(nxt, b)
            return carry

        lax.fori_loop(0, NFULL + 1, body, 0)
        for b in range(NCHUNK - (NFULL + 1) * NBUF):
            g = (NFULL + 1) * NBUF + b
            wait_gather(g, b)
            fire_scatter(g, b)
            wait_scatter(g, b)

    @pl.when(c == 0)
    def _():
        run(sup0_hbm)

    @pl.when(c == 1)
    def _():
        run(sup1_hbm)

    plsc.subcore_barrier()

    r0 = s * ROWS_PER_TILE
    pltpu.sync_copy(accum.at[pl.ds(r0, ROWS_PER_TILE)],
                    out_hbm.at[pl.ds(r0, ROWS_PER_TILE), pl.ds(c * DH, DH)])


def _scatter(ei3, sup0, sup1, bias):
    mesh = plsc.VectorSubcoreMesh(core_axis_name="c", subcore_axis_name="s")
    k = functools.partial(
        pl.kernel,
        out_type=jax.ShapeDtypeStruct((N, D), jnp.float32),
        mesh=mesh,
        scratch_types=[
            pltpu.VMEM((NCHUNK, CHUNK), jnp.int32),
            pltpu.VMEM((NCHUNK, CHUNK), jnp.int32),
            pltpu.VMEM((D,), jnp.float32),
            pltpu.VMEM((NBUF, CHUNK, DH), jnp.float32),
            pltpu.VMEM((ZROWS, DH), jnp.float32),
            pltpu.VMEM_SHARED((N, DH), jnp.float32),
            pltpu.SemaphoreType.DMA((NBUF,)),
            pltpu.SemaphoreType.DMA((NBUF,)),
        ],
        compiler_params=pltpu.CompilerParams(use_tc_tiling_on_sc=False),
    )(_scatter_body)
    return k(ei3, sup0, sup1, bias)


def kernel(edge_index, input_feature, weight, bias):
    sup0, sup1 = _matmul(input_feature, weight)
    return _scatter(edge_index.reshape(2, NS * NCHUNK, CHUNK),
                    sup0, sup1, bias)
